# baseline (speedup 1.0000x reference)
import functools

import jax
import jax.numpy as jnp
from jax.experimental import pallas as pl
from jax.experimental.pallas import tpu as pltpu

_BM = 4096


def _body(n_classes, codes_ref, cents_ref, out_ref, cents_nb):
    c = n_classes

    @pl.when(pl.program_id(0) == 0)
    def _():
        cr = cents_ref[...]
        for kk in range(4):
            ck = cr[:, kk, :]
            cinv = jax.lax.rsqrt(
                jnp.maximum(jnp.sum(ck * ck, axis=1, keepdims=True), 1e-24))
            cents_nb[kk * c:(kk + 1) * c, :] = (ck * cinv).astype(jnp.bfloat16)

    codes = codes_ref[...]
    inv = jax.lax.rsqrt(
        jnp.maximum(jnp.sum(codes * codes, axis=1, keepdims=True), 1e-24))
    codes_n = (codes * inv).astype(jnp.bfloat16)

    dn = (((1,), (1,)), ((), ()))
    m = jax.lax.dot_general(codes_n, cents_nb[0 * c:1 * c, :], dn,
                            preferred_element_type=jnp.float32)
    for kk in range(1, 4):
        m = jnp.maximum(m, jax.lax.dot_general(
            codes_n, cents_nb[kk * c:(kk + 1) * c, :], dn,
            preferred_element_type=jnp.float32))
    out_ref[...] = 1.0 - m


def kernel(codes, centroids):
    b, d = codes.shape
    c, k, _ = centroids.shape
    n_steps = b // _BM
    body = functools.partial(_body, c)
    return pl.pallas_call(
        body,
        grid=(n_steps,),
        in_specs=[
            pl.BlockSpec((_BM, d), lambda i: (i, 0)),
            pl.BlockSpec((c, k, d), lambda i: (0, 0, 0)),
        ],
        out_specs=pl.BlockSpec((_BM, c), lambda i: (i, 0)),
        out_shape=jax.ShapeDtypeStruct((b, c), jnp.float32),
        scratch_shapes=[pltpu.VMEM((k * c, d), jnp.bfloat16)],
        compiler_params=pltpu.CompilerParams(
            allow_input_fusion=[False, True]),
    )(codes, centroids)


# speedup vs baseline: 1.0324x; 1.0324x over previous
import functools

import jax
import jax.numpy as jnp
from jax.experimental import pallas as pl
from jax.experimental.pallas import tpu as pltpu

_BM = 4096


def _body(n_classes, codes_ref, cents_ref, out_ref, cents_nb):
    c = n_classes

    @pl.when(pl.program_id(0) == 0)
    def _():
        cf = cents_ref[...].astype(jnp.float32)
        cinv = jax.lax.rsqrt(
            jnp.maximum(jnp.sum(cf * cf, axis=1, keepdims=True), 1e-24))
        cents_nb[...] = (cf * cinv).astype(jnp.bfloat16)

    codes = codes_ref[...]
    inv = jax.lax.rsqrt(
        jnp.maximum(jnp.sum(codes * codes, axis=1, keepdims=True), 1e-24))
    codes_n = (codes * inv).astype(jnp.bfloat16)

    dn = (((1,), (1,)), ((), ()))
    m = jax.lax.dot_general(codes_n, cents_nb[0 * c:1 * c, :], dn,
                            preferred_element_type=jnp.float32)
    for kk in range(1, 4):
        m = jnp.maximum(m, jax.lax.dot_general(
            codes_n, cents_nb[kk * c:(kk + 1) * c, :], dn,
            preferred_element_type=jnp.float32))
    out_ref[...] = 1.0 - m


def kernel(codes, centroids):
    b, d = codes.shape
    c, k, _ = centroids.shape
    cents_t = centroids.transpose(1, 0, 2).reshape(k * c, d).astype(
        jnp.bfloat16)
    n_steps = b // _BM
    body = functools.partial(_body, c)
    return pl.pallas_call(
        body,
        grid=(n_steps,),
        in_specs=[
            pl.BlockSpec((_BM, d), lambda i: (i, 0)),
            pl.BlockSpec((k * c, d), lambda i: (0, 0)),
        ],
        out_specs=pl.BlockSpec((_BM, c), lambda i: (i, 0)),
        out_shape=jax.ShapeDtypeStruct((b, c), jnp.float32),
        scratch_shapes=[pltpu.VMEM((k * c, d), jnp.bfloat16)],
    )(codes, cents_t)
